# Initial kernel scaffold; baseline (speedup 1.0000x reference)
import jax
import jax.numpy as jnp
from jax.experimental import pallas as pl


def kernel(featmap, gt):
    raise NotImplementedError("write your pallas kernel here")


# speedup vs baseline: 1.1833x; 1.1833x over previous
import functools

import jax
import jax.numpy as jnp
from jax import lax
from jax.experimental import pallas as pl
from jax.experimental.pallas import tpu as pltpu
from jax.experimental.pallas import tpu_sc as plsc

_B, _C, _H, _W = 4, 32, 224, 224
_P = _H * _W
_NS = 16
_PX = _P // _NS
_NBLK = _PX // 16
_NL = 9
_ACC = _NL * _C
_ROW = 16 + 16 + _ACC
_MARGIN_VAR = 0.5
_MARGIN_DIST = 3.0


def _vsqrt(x):
    i = plsc.bitcast(x, jnp.int32)
    y = plsc.bitcast(lax.shift_right_arithmetic(i, 1) + 0x1FBD1DF5, jnp.float32)
    y = 0.5 * (y + x / y)
    y = 0.5 * (y + x / y)
    return jnp.where(x > 0.0, y, 0.0)


def _sc_body(feat_hbm, gt_hbm, out_hbm, feat_v, lab_v, acc2_v, cnt2_v,
             pull2_v, part_v, red_v, cen_v, rcp_v, pullp_v, pullred_v,
             outbuf_v, sh_part, sh_pull):
    cid = lax.axis_index("c")
    sid = lax.axis_index("s")
    lane = lax.iota(jnp.int32, 16)
    zero16 = jnp.zeros((16,), jnp.float32)
    one16 = jnp.ones((16,), jnp.float32)
    bank = lane * _ACC
    bankc = lane * 16

    for b in range(_B):
        @pl.when(cid == (b // 2))
        def _process():
            base = sid * _PX
            pltpu.sync_copy(feat_hbm.at[b, :, pl.ds(base, _PX)], feat_v)
            pltpu.sync_copy(gt_hbm.at[b, pl.ds(base, _PX)], lab_v)

            def _zero_acc(i, carry):
                acc2_v[pl.ds(i * 16, 16)] = zero16
                return carry
            lax.fori_loop(0, _ACC * 16 // 16, _zero_acc, 0)
            for i in range(16):
                cnt2_v[pl.ds(i * 16, 16)] = zero16
                pull2_v[pl.ds(i * 16, 16)] = zero16

            def _p1(blk, carry):
                lab = lab_v[pl.ds(blk * 16, 16)]
                ib = bank + lab * _C
                for c in range(_C):
                    v = feat_v[c, pl.ds(blk * 16, 16)]
                    plsc.addupdate_scatter(acc2_v, [ib + c], v)
                plsc.addupdate_scatter(cnt2_v, [bankc + lab], one16)
                return carry
            lax.fori_loop(0, _NBLK, _p1, 0)

            for j in range(_ACC // 16):
                s = acc2_v[pl.ds(j * 16, 16)]
                for t in range(1, 16):
                    s = s + acc2_v[pl.ds(t * _ACC + j * 16, 16)]
                part_v[pl.ds(j * 16, 16)] = s
            s = cnt2_v[pl.ds(0, 16)]
            for t in range(1, 16):
                s = s + cnt2_v[pl.ds(t * 16, 16)]
            part_v[pl.ds(_ACC, 16)] = s
            part_v[pl.ds(_ACC + 16, 16)] = zero16

            pltpu.sync_copy(part_v, sh_part.at[sid])
            plsc.subcore_barrier()
            pltpu.sync_copy(sh_part, red_v)

            cnt = red_v[0, pl.ds(_ACC, 16)]
            for t in range(1, 16):
                cnt = cnt + red_v[t, pl.ds(_ACC, 16)]
            rcp_v[...] = 1.0 / jnp.maximum(cnt, 1.0)
            for l in range(_NL):
                rl = plsc.load_gather(rcp_v, [jnp.full((16,), l, jnp.int32)])
                for h in range(2):
                    off = l * _C + h * 16
                    s = red_v[0, pl.ds(off, 16)]
                    for t in range(1, 16):
                        s = s + red_v[t, pl.ds(off, 16)]
                    cen_v[pl.ds(off, 16)] = s * rl

            def _p2(blk, carry):
                lab = lab_v[pl.ds(blk * 16, 16)]
                i0 = lab * _C
                dsq = zero16
                for c in range(_C):
                    v = feat_v[c, pl.ds(blk * 16, 16)]
                    ctr = plsc.load_gather(cen_v, [i0 + c])
                    d = v - ctr
                    dsq = dsq + d * d
                loss = jnp.maximum(_vsqrt(dsq) - _MARGIN_VAR, 0.0)
                plsc.addupdate_scatter(pull2_v, [bankc + lab], loss)
                return carry
            lax.fori_loop(0, _NBLK, _p2, 0)

            ps = pull2_v[pl.ds(0, 16)]
            for t in range(1, 16):
                ps = ps + pull2_v[pl.ds(t * 16, 16)]
            pullp_v[...] = ps
            pltpu.sync_copy(pullp_v, sh_pull.at[sid])
            plsc.subcore_barrier()

            @pl.when(sid == 0)
            def _emit():
                pltpu.sync_copy(sh_pull, pullred_v)
                q = pullred_v[0, pl.ds(0, 16)]
                for t in range(1, 16):
                    q = q + pullred_v[t, pl.ds(0, 16)]
                outbuf_v[pl.ds(0, 16)] = q
                outbuf_v[pl.ds(16, 16)] = cnt
                for j in range(_ACC // 16):
                    outbuf_v[pl.ds(32 + j * 16, 16)] = cen_v[pl.ds(j * 16, 16)]
                pltpu.sync_copy(outbuf_v, out_hbm.at[b])


_sc_summarize = functools.partial(
    pl.kernel,
    out_type=jax.ShapeDtypeStruct((_B, _ROW), jnp.float32),
    mesh=plsc.VectorSubcoreMesh(core_axis_name="c", subcore_axis_name="s"),
    compiler_params=pltpu.CompilerParams(
        use_tc_tiling_on_sc=False, needs_layout_passes=False),
    scratch_types=[
        pltpu.VMEM((_C, _PX), jnp.float32),
        pltpu.VMEM((_PX,), jnp.int32),
        pltpu.VMEM((_ACC * 16,), jnp.float32),
        pltpu.VMEM((256,), jnp.float32),
        pltpu.VMEM((256,), jnp.float32),
        pltpu.VMEM((_ROW,), jnp.float32),
        pltpu.VMEM((_NS, _ROW), jnp.float32),
        pltpu.VMEM((_ACC,), jnp.float32),
        pltpu.VMEM((16,), jnp.float32),
        pltpu.VMEM((16,), jnp.float32),
        pltpu.VMEM((_NS, 16), jnp.float32),
        pltpu.VMEM((_ROW,), jnp.float32),
        pltpu.VMEM_SHARED((_NS, _ROW), jnp.float32),
        pltpu.VMEM_SHARED((_NS, 16), jnp.float32),
    ],
)(_sc_body)


def _tc_combine(pm_ref, cm_ref, cen_ref, pres_c_ref, pres_r_ref, out_ref):
    pm = pm_ref[...]
    cm = cm_ref[...]
    cen = cen_ref[...]
    lanes = lax.broadcasted_iota(jnp.int32, (_B, 16), 1)
    valid = (lanes >= 1) & (lanes <= _NL - 1) & (cm > 0.0)
    vals = pm / jnp.maximum(cm, 1.0)
    pn = jnp.sum(jnp.where(valid, vals, 0.0))
    pc = jnp.sum(valid.astype(jnp.float32))

    diff = cen[:, None, :] - cen[None, :, :]
    d2 = jnp.sum(diff * diff, axis=-1)
    pv = jnp.maximum(2.0 * _MARGIN_DIST - jnp.sqrt(d2), 0.0)
    ii = lax.broadcasted_iota(jnp.int32, (_B * _NL, _B * _NL), 0)
    jj = lax.broadcasted_iota(jnp.int32, (_B * _NL, _B * _NL), 1)
    m = ((ii // _NL == jj // _NL) & (ii != jj)
         & (ii % _NL != 0) & (jj % _NL != 0))
    w = m & (pres_c_ref[...] > 0.0) & (pres_r_ref[...] > 0.0)
    qn = jnp.sum(jnp.where(w, pv, 0.0))
    qc = jnp.sum(w.astype(jnp.float32))

    pull = jnp.where(pc > 0.0, pn / jnp.maximum(pc, 1.0), 0.0)
    push = jnp.where(qc > 0.0, qn / jnp.maximum(qc, 1.0), 0.0)
    out_ref[...] = jnp.reshape(pull + push, (1, 1))


def kernel(featmap, gt):
    feat = featmap.reshape(_B, _C, _P)
    lab = gt.reshape(_B, _P)
    summary = _sc_summarize(feat, lab)
    pm = summary[:, 0:16]
    cm = summary[:, 16:32]
    cen = summary[:, 32:].reshape(_B * _NL, _C)
    cnt36 = summary[:, 16:16 + _NL].reshape(_B * _NL, 1)
    out = pl.pallas_call(
        _tc_combine,
        out_shape=jax.ShapeDtypeStruct((1, 1), jnp.float32),
    )(pm, cm, cen, cnt36, cnt36.reshape(1, _B * _NL))
    return out[0, 0]
